# baseline (speedup 1.0000x reference)
import functools

import jax
import jax.numpy as jnp
from jax import lax
from jax.experimental import pallas as pl
from jax.experimental.pallas import tpu as pltpu
from jax.experimental.pallas import tpu_sc as plsc

N_NODES = 10000
D = 128
N_EDGES = 320000

NC = 2
NS = 16
NW = NC * NS

R_PAD = 10240
E_PAD = 327680
CHUNK = 128
GROUP = 8
CPW0 = 104
CPW1 = 56
NBUF = 2
E_ROWS = E_PAD // CHUNK
ROWS_PER_TILE = R_PAD // NS
INV_ROWS = R_PAD // NW
DEG_ROWS = E_ROWS // NS

_mesh = plsc.VectorSubcoreMesh(core_axis_name="c", subcore_axis_name="s")


@functools.partial(
    pl.kernel,
    out_type=[jax.ShapeDtypeStruct((R_PAD, D), jnp.float32),
              jax.ShapeDtypeStruct((NS, R_PAD), jnp.float32)],
    mesh=_mesh,
    compiler_params=pltpu.CompilerParams(needs_layout_passes=False),
    scratch_types=[
        pltpu.VMEM((R_PAD,), jnp.float32),
        pltpu.VMEM((DEG_ROWS, CHUNK), jnp.int32),
        pltpu.VMEM((NS, ROWS_PER_TILE), jnp.float32),
        pltpu.VMEM((ROWS_PER_TILE,), jnp.float32),
        pltpu.VMEM((INV_ROWS, D), jnp.float32),
    ],
)
def _deg_kernel(dst_hbm, zcnt_hbm, invb_hbm, degsh,
                cnt, dstv, deg16, totbuf, rowbuf):
    cid = lax.axis_index("c")
    sid = lax.axis_index("s")

    pltpu.sync_copy(zcnt_hbm, cnt)
    pltpu.sync_copy(dst_hbm.at[pl.ds(sid * DEG_ROWS, DEG_ROWS)], dstv)

    ones = jnp.ones((16,), jnp.float32)

    def count_body(i, carry):
        r = i // 8
        j = i % 8
        idx = dstv[r, pl.ds(j * 16, 16)]
        plsc.addupdate_scatter(cnt, [idx], ones)
        return carry

    lax.fori_loop(0, DEG_ROWS * 8, count_body, 0)

    pltpu.sync_copy(cnt, degsh.at[sid])
    plsc.subcore_barrier()

    base = sid * ROWS_PER_TILE
    for r in range(NS):
        pltpu.sync_copy(degsh.at[r, pl.ds(base, ROWS_PER_TILE)], deg16.at[r])

    def sum_body(j, carry):
        tot = deg16[0, pl.ds(j * 16, 16)]
        for r in range(1, NS):
            tot = tot + deg16[r, pl.ds(j * 16, 16)]
        totbuf[pl.ds(j * 16, 16)] = tot
        return carry

    lax.fori_loop(0, ROWS_PER_TILE // 16, sum_body, 0)

    @pl.when(cid == 0)
    def _():
        for half in range(2):
            def inv_body(g, carry):
                d16 = totbuf[pl.ds(half * INV_ROWS + g * 16, 16)]
                vinv = 1.0 / jnp.maximum(d16, 1.0)
                for l in range(16):
                    bc = jnp.full((16,), vinv[l], jnp.float32)
                    for j in range(8):
                        rowbuf[g * 16 + l, pl.ds(j * 16, 16)] = bc
                return carry

            lax.fori_loop(0, INV_ROWS // 16, inv_body, 0)
            pltpu.sync_copy(
                rowbuf,
                invb_hbm.at[pl.ds(base + half * INV_ROWS, INV_ROWS)])


@functools.partial(
    pl.kernel,
    out_type=jax.ShapeDtypeStruct((NC, R_PAD, D), jnp.float32),
    mesh=_mesh,
    scratch_types=[
        pltpu.VMEM((GROUP, CHUNK), jnp.int32),
        pltpu.VMEM((GROUP, CHUNK), jnp.int32),
        [pltpu.VMEM((CHUNK, D), jnp.float32) for _ in range(NBUF)],
        [pltpu.SemaphoreType.DMA for _ in range(NBUF)],
        pltpu.VMEM_SHARED((R_PAD, D), jnp.float32),
    ],
)
def _agg_kernel(h_hbm, src_hbm, dst_hbm, zrow_hbm, part_hbm,
                sidx, didx, bufs, gsems, acc):
    cid = lax.axis_index("c")
    sid = lax.axis_index("s")

    my_cpw = jnp.where(cid == 0, CPW0, CPW1)
    row_base = jnp.where(cid == 0, sid * CPW0, NS * CPW0 + sid * CPW1)

    pltpu.sync_copy(zrow_hbm, acc.at[pl.ds(sid * ROWS_PER_TILE, ROWS_PER_TILE)])
    plsc.subcore_barrier()

    def start_gather(c, k):
        pltpu.async_copy(h_hbm.at[sidx.at[c]], bufs[k], gsems[k])

    def wait_gather(k):
        pltpu.make_async_copy(h_hbm.at[pl.ds(0, CHUNK)], bufs[k], gsems[k]).wait()

    def group_body(g, carry):
        row0 = row_base + g * GROUP
        pltpu.sync_copy(src_hbm.at[pl.ds(row0, GROUP)], sidx)
        pltpu.sync_copy(dst_hbm.at[pl.ds(row0, GROUP)], didx)
        start_gather(0, 0)

        def body(i, carry):
            c0 = NBUF * i
            for k in range(NBUF):
                c = c0 + k

                @pl.when(c + 1 < GROUP)
                def _(c=c, k=k):
                    start_gather(c + 1, (k + 1) % NBUF)

                wait_gather(k)
                pltpu.sync_copy(bufs[k], acc.at[didx.at[c]], add=True)
            return carry

        lax.fori_loop(0, GROUP // NBUF, body, 0)
        return carry

    lax.fori_loop(0, my_cpw // GROUP, group_body, 0)
    plsc.subcore_barrier()

    pltpu.sync_copy(
        acc.at[pl.ds(sid * ROWS_PER_TILE, ROWS_PER_TILE)],
        part_hbm.at[cid, pl.ds(sid * ROWS_PER_TILE, ROWS_PER_TILE)],
    )


def _tc_layer_body(last, h, p, invb, ws, wn, b, out):
    hn = (p[0] + p[1]) * invb[...]
    y = (
        jnp.dot(h[...], ws[...], preferred_element_type=jnp.float32)
        + jnp.dot(hn, wn[...], preferred_element_type=jnp.float32)
        + b[...]
    )
    y = jnp.maximum(y, 0.0)
    if last:
        nrm = jnp.sqrt(jnp.sum(y * y, axis=1, keepdims=True))
        y = y / jnp.maximum(nrm, 1e-12)
    out[...] = y


def _tc_layer(h, p, invb, ws, wn, b, last):
    blk = 1000
    grid = (N_NODES // blk,)
    return pl.pallas_call(
        functools.partial(_tc_layer_body, last),
        grid=grid,
        in_specs=[
            pl.BlockSpec((blk, D), lambda i: (i, 0)),
            pl.BlockSpec((NC, blk, D), lambda i: (0, i, 0)),
            pl.BlockSpec((blk, D), lambda i: (i, 0)),
            pl.BlockSpec((D, D), lambda i: (0, 0)),
            pl.BlockSpec((D, D), lambda i: (0, 0)),
            pl.BlockSpec((1, D), lambda i: (0, 0)),
        ],
        out_specs=pl.BlockSpec((blk, D), lambda i: (i, 0)),
        out_shape=jax.ShapeDtypeStruct((N_NODES, D), jnp.float32),
    )(h, p, invb, ws, wn, b)


@jax.jit
def kernel(x, edge_index, W_self1, W_neigh1, b1, W_self2, W_neigh2, b2):
    src = edge_index[0].astype(jnp.int32)
    dst = edge_index[1].astype(jnp.int32)
    pad = E_PAD - N_EDGES
    src_p = jnp.concatenate(
        [src, jnp.zeros((pad,), jnp.int32)]).reshape(E_ROWS, CHUNK)
    dst_p = jnp.concatenate(
        [dst, jnp.full((pad,), N_NODES, jnp.int32)]).reshape(E_ROWS, CHUNK)
    zcnt = jnp.zeros((R_PAD,), jnp.float32)
    zrow = jnp.zeros((ROWS_PER_TILE, D), jnp.float32)

    invb, _ = _deg_kernel(dst_p, zcnt)
    b1r = b1.reshape(1, D)
    b2r = b2.reshape(1, D)

    p1 = _agg_kernel(x, src_p, dst_p, zrow)
    h1 = _tc_layer(x, p1, invb, W_self1, W_neigh1, b1r, last=False)
    p2 = _agg_kernel(h1, src_p, dst_p, zrow)
    return _tc_layer(h1, p2, invb, W_self2, W_neigh2, b2r, last=True)


# speedup vs baseline: 1.0629x; 1.0629x over previous
import functools

import jax
import jax.numpy as jnp
from jax import lax
from jax.experimental import pallas as pl
from jax.experimental.pallas import tpu as pltpu
from jax.experimental.pallas import tpu_sc as plsc

N_NODES = 10000
D = 128
N_EDGES = 320000

NC = 2
NS = 16
NW = NC * NS

R_PAD = 10240
E_PAD = 327680
CHUNK = 128
GROUP = 8
CPW0 = 136
CPW1 = 24
NBUF = 2
E_ROWS = E_PAD // CHUNK
ROWS_PER_TILE = R_PAD // NS
INV_ROWS = R_PAD // NW
DEG_ROWS = E_ROWS // NS

_mesh = plsc.VectorSubcoreMesh(core_axis_name="c", subcore_axis_name="s")


@functools.partial(
    pl.kernel,
    out_type=[jax.ShapeDtypeStruct((R_PAD, D), jnp.float32),
              jax.ShapeDtypeStruct((NS, R_PAD), jnp.float32)],
    mesh=_mesh,
    compiler_params=pltpu.CompilerParams(needs_layout_passes=False),
    scratch_types=[
        pltpu.VMEM((R_PAD,), jnp.float32),
        pltpu.VMEM((DEG_ROWS, CHUNK), jnp.int32),
        pltpu.VMEM((NS, ROWS_PER_TILE), jnp.float32),
        pltpu.VMEM((ROWS_PER_TILE,), jnp.float32),
        pltpu.VMEM((INV_ROWS, D), jnp.float32),
    ],
)
def _deg_kernel(dst_hbm, zcnt_hbm, invb_hbm, degsh,
                cnt, dstv, deg16, totbuf, rowbuf):
    cid = lax.axis_index("c")
    sid = lax.axis_index("s")

    pltpu.sync_copy(zcnt_hbm, cnt)
    pltpu.sync_copy(dst_hbm.at[pl.ds(sid * DEG_ROWS, DEG_ROWS)], dstv)

    ones = jnp.ones((16,), jnp.float32)

    def count_body(i, carry):
        r = i // 8
        j = i % 8
        idx = dstv[r, pl.ds(j * 16, 16)]
        plsc.addupdate_scatter(cnt, [idx], ones)
        return carry

    lax.fori_loop(0, DEG_ROWS * 8, count_body, 0)

    pltpu.sync_copy(cnt, degsh.at[sid])
    plsc.subcore_barrier()

    base = sid * ROWS_PER_TILE
    for r in range(NS):
        pltpu.sync_copy(degsh.at[r, pl.ds(base, ROWS_PER_TILE)], deg16.at[r])

    def sum_body(j, carry):
        tot = deg16[0, pl.ds(j * 16, 16)]
        for r in range(1, NS):
            tot = tot + deg16[r, pl.ds(j * 16, 16)]
        totbuf[pl.ds(j * 16, 16)] = tot
        return carry

    lax.fori_loop(0, ROWS_PER_TILE // 16, sum_body, 0)

    @pl.when(cid == 0)
    def _():
        for half in range(2):
            def inv_body(g, carry):
                d16 = totbuf[pl.ds(half * INV_ROWS + g * 16, 16)]
                vinv = 1.0 / jnp.maximum(d16, 1.0)
                for l in range(16):
                    bc = jnp.full((16,), vinv[l], jnp.float32)
                    for j in range(8):
                        rowbuf[g * 16 + l, pl.ds(j * 16, 16)] = bc
                return carry

            lax.fori_loop(0, INV_ROWS // 16, inv_body, 0)
            pltpu.sync_copy(
                rowbuf,
                invb_hbm.at[pl.ds(base + half * INV_ROWS, INV_ROWS)])


@functools.partial(
    pl.kernel,
    out_type=jax.ShapeDtypeStruct((NC, R_PAD, D), jnp.float32),
    mesh=_mesh,
    scratch_types=[
        pltpu.VMEM((GROUP, CHUNK), jnp.int32),
        pltpu.VMEM((GROUP, CHUNK), jnp.int32),
        [pltpu.VMEM((CHUNK, D), jnp.float32) for _ in range(NBUF)],
        [pltpu.SemaphoreType.DMA for _ in range(NBUF)],
        pltpu.VMEM_SHARED((R_PAD, D), jnp.float32),
    ],
)
def _agg_kernel(h_hbm, src_hbm, dst_hbm, zrow_hbm, part_hbm,
                sidx, didx, bufs, gsems, acc):
    cid = lax.axis_index("c")
    sid = lax.axis_index("s")

    my_cpw = jnp.where(cid == 0, CPW0, CPW1)
    row_base = jnp.where(cid == 0, sid * CPW0, NS * CPW0 + sid * CPW1)

    pltpu.sync_copy(zrow_hbm, acc.at[pl.ds(sid * ROWS_PER_TILE, ROWS_PER_TILE)])
    plsc.subcore_barrier()

    def start_gather(c, k):
        pltpu.async_copy(h_hbm.at[sidx.at[c]], bufs[k], gsems[k])

    def wait_gather(k):
        pltpu.make_async_copy(h_hbm.at[pl.ds(0, CHUNK)], bufs[k], gsems[k]).wait()

    def group_body(g, carry):
        row0 = row_base + g * GROUP
        pltpu.sync_copy(src_hbm.at[pl.ds(row0, GROUP)], sidx)
        pltpu.sync_copy(dst_hbm.at[pl.ds(row0, GROUP)], didx)
        start_gather(0, 0)

        def body(i, carry):
            c0 = NBUF * i
            for k in range(NBUF):
                c = c0 + k

                @pl.when(c + 1 < GROUP)
                def _(c=c, k=k):
                    start_gather(c + 1, (k + 1) % NBUF)

                wait_gather(k)
                pltpu.sync_copy(bufs[k], acc.at[didx.at[c]], add=True)
            return carry

        lax.fori_loop(0, GROUP // NBUF, body, 0)
        return carry

    lax.fori_loop(0, my_cpw // GROUP, group_body, 0)
    plsc.subcore_barrier()

    pltpu.sync_copy(
        acc.at[pl.ds(sid * ROWS_PER_TILE, ROWS_PER_TILE)],
        part_hbm.at[cid, pl.ds(sid * ROWS_PER_TILE, ROWS_PER_TILE)],
    )


def _tc_layer_body(last, h, p, invb, ws, wn, b, out):
    hn = (p[0] + p[1]) * invb[...]
    y = (
        jnp.dot(h[...], ws[...], preferred_element_type=jnp.float32)
        + jnp.dot(hn, wn[...], preferred_element_type=jnp.float32)
        + b[...]
    )
    y = jnp.maximum(y, 0.0)
    if last:
        nrm = jnp.sqrt(jnp.sum(y * y, axis=1, keepdims=True))
        y = y / jnp.maximum(nrm, 1e-12)
    out[...] = y


def _tc_layer(h, p, invb, ws, wn, b, last):
    blk = 1000
    grid = (N_NODES // blk,)
    return pl.pallas_call(
        functools.partial(_tc_layer_body, last),
        grid=grid,
        in_specs=[
            pl.BlockSpec((blk, D), lambda i: (i, 0)),
            pl.BlockSpec((NC, blk, D), lambda i: (0, i, 0)),
            pl.BlockSpec((blk, D), lambda i: (i, 0)),
            pl.BlockSpec((D, D), lambda i: (0, 0)),
            pl.BlockSpec((D, D), lambda i: (0, 0)),
            pl.BlockSpec((1, D), lambda i: (0, 0)),
        ],
        out_specs=pl.BlockSpec((blk, D), lambda i: (i, 0)),
        out_shape=jax.ShapeDtypeStruct((N_NODES, D), jnp.float32),
    )(h, p, invb, ws, wn, b)


@jax.jit
def kernel(x, edge_index, W_self1, W_neigh1, b1, W_self2, W_neigh2, b2):
    src = edge_index[0].astype(jnp.int32)
    dst = edge_index[1].astype(jnp.int32)
    pad = E_PAD - N_EDGES
    src_p = jnp.concatenate(
        [src, jnp.zeros((pad,), jnp.int32)]).reshape(E_ROWS, CHUNK)
    dst_p = jnp.concatenate(
        [dst, jnp.full((pad,), N_NODES, jnp.int32)]).reshape(E_ROWS, CHUNK)
    zcnt = jnp.zeros((R_PAD,), jnp.float32)
    zrow = jnp.zeros((ROWS_PER_TILE, D), jnp.float32)

    invb, _ = _deg_kernel(dst_p, zcnt)
    b1r = b1.reshape(1, D)
    b2r = b2.reshape(1, D)

    p1 = _agg_kernel(x, src_p, dst_p, zrow)
    h1 = _tc_layer(x, p1, invb, W_self1, W_neigh1, b1r, last=False)
    p2 = _agg_kernel(h1, src_p, dst_p, zrow)
    return _tc_layer(h1, p2, invb, W_self2, W_neigh2, b2r, last=True)


# speedup vs baseline: 1.1130x; 1.0472x over previous
import functools

import jax
import jax.numpy as jnp
from jax import lax
from jax.experimental import pallas as pl
from jax.experimental.pallas import tpu as pltpu
from jax.experimental.pallas import tpu_sc as plsc

N_NODES = 10000
D = 128
N_EDGES = 320000

NC = 2
NS = 16
NW = NC * NS

R_PAD = 10240
E_PAD = 327680
CHUNK = 128
GROUP = 8
CPW0 = 144
CPW1 = 16
NBUF = 2
E_ROWS = E_PAD // CHUNK
ROWS_PER_TILE = R_PAD // NS
INV_ROWS = R_PAD // NW
DEG_ROWS = E_ROWS // NS

_mesh = plsc.VectorSubcoreMesh(core_axis_name="c", subcore_axis_name="s")


@functools.partial(
    pl.kernel,
    out_type=[jax.ShapeDtypeStruct((R_PAD, D), jnp.float32),
              jax.ShapeDtypeStruct((NS, R_PAD), jnp.float32)],
    mesh=_mesh,
    compiler_params=pltpu.CompilerParams(needs_layout_passes=False),
    scratch_types=[
        pltpu.VMEM((R_PAD,), jnp.float32),
        pltpu.VMEM((DEG_ROWS, CHUNK), jnp.int32),
        pltpu.VMEM((NS, ROWS_PER_TILE), jnp.float32),
        pltpu.VMEM((ROWS_PER_TILE,), jnp.float32),
        pltpu.VMEM((INV_ROWS, D), jnp.float32),
    ],
)
def _deg_kernel(dst_hbm, zcnt_hbm, invb_hbm, degsh,
                cnt, dstv, deg16, totbuf, rowbuf):
    cid = lax.axis_index("c")
    sid = lax.axis_index("s")

    pltpu.sync_copy(zcnt_hbm, cnt)
    pltpu.sync_copy(dst_hbm.at[pl.ds(sid * DEG_ROWS, DEG_ROWS)], dstv)

    ones = jnp.ones((16,), jnp.float32)

    def count_body(i, carry):
        r = i // 8
        j = i % 8
        idx = dstv[r, pl.ds(j * 16, 16)]
        plsc.addupdate_scatter(cnt, [idx], ones)
        return carry

    lax.fori_loop(0, DEG_ROWS * 8, count_body, 0)

    pltpu.sync_copy(cnt, degsh.at[sid])
    plsc.subcore_barrier()

    base = sid * ROWS_PER_TILE
    for r in range(NS):
        pltpu.sync_copy(degsh.at[r, pl.ds(base, ROWS_PER_TILE)], deg16.at[r])

    def sum_body(j, carry):
        tot = deg16[0, pl.ds(j * 16, 16)]
        for r in range(1, NS):
            tot = tot + deg16[r, pl.ds(j * 16, 16)]
        totbuf[pl.ds(j * 16, 16)] = tot
        return carry

    lax.fori_loop(0, ROWS_PER_TILE // 16, sum_body, 0)

    @pl.when(cid == 0)
    def _():
        for half in range(2):
            def inv_body(g, carry):
                d16 = totbuf[pl.ds(half * INV_ROWS + g * 16, 16)]
                vinv = 1.0 / jnp.maximum(d16, 1.0)
                for l in range(16):
                    bc = jnp.full((16,), vinv[l], jnp.float32)
                    for j in range(8):
                        rowbuf[g * 16 + l, pl.ds(j * 16, 16)] = bc
                return carry

            lax.fori_loop(0, INV_ROWS // 16, inv_body, 0)
            pltpu.sync_copy(
                rowbuf,
                invb_hbm.at[pl.ds(base + half * INV_ROWS, INV_ROWS)])


@functools.partial(
    pl.kernel,
    out_type=jax.ShapeDtypeStruct((NC, R_PAD, D), jnp.float32),
    mesh=_mesh,
    scratch_types=[
        pltpu.VMEM((GROUP, CHUNK), jnp.int32),
        pltpu.VMEM((GROUP, CHUNK), jnp.int32),
        [pltpu.VMEM((CHUNK, D), jnp.float32) for _ in range(NBUF)],
        [pltpu.SemaphoreType.DMA for _ in range(NBUF)],
        pltpu.VMEM_SHARED((R_PAD, D), jnp.float32),
    ],
)
def _agg_kernel(h_hbm, src_hbm, dst_hbm, zrow_hbm, part_hbm,
                sidx, didx, bufs, gsems, acc):
    cid = lax.axis_index("c")
    sid = lax.axis_index("s")

    my_cpw = jnp.where(cid == 0, CPW0, CPW1)
    row_base = jnp.where(cid == 0, sid * CPW0, NS * CPW0 + sid * CPW1)

    pltpu.sync_copy(zrow_hbm, acc.at[pl.ds(sid * ROWS_PER_TILE, ROWS_PER_TILE)])
    plsc.subcore_barrier()

    def start_gather(c, k):
        pltpu.async_copy(h_hbm.at[sidx.at[c]], bufs[k], gsems[k])

    def wait_gather(k):
        pltpu.make_async_copy(h_hbm.at[pl.ds(0, CHUNK)], bufs[k], gsems[k]).wait()

    def group_body(g, carry):
        row0 = row_base + g * GROUP
        pltpu.sync_copy(src_hbm.at[pl.ds(row0, GROUP)], sidx)
        pltpu.sync_copy(dst_hbm.at[pl.ds(row0, GROUP)], didx)
        start_gather(0, 0)

        def body(i, carry):
            c0 = NBUF * i
            for k in range(NBUF):
                c = c0 + k

                @pl.when(c + 1 < GROUP)
                def _(c=c, k=k):
                    start_gather(c + 1, (k + 1) % NBUF)

                wait_gather(k)
                pltpu.sync_copy(bufs[k], acc.at[didx.at[c]], add=True)
            return carry

        lax.fori_loop(0, GROUP // NBUF, body, 0)
        return carry

    lax.fori_loop(0, my_cpw // GROUP, group_body, 0)
    plsc.subcore_barrier()

    pltpu.sync_copy(
        acc.at[pl.ds(sid * ROWS_PER_TILE, ROWS_PER_TILE)],
        part_hbm.at[cid, pl.ds(sid * ROWS_PER_TILE, ROWS_PER_TILE)],
    )


def _tc_layer_body(last, h, p, invb, ws, wn, b, out):
    hn = (p[0] + p[1]) * invb[...]
    y = (
        jnp.dot(h[...], ws[...], preferred_element_type=jnp.float32)
        + jnp.dot(hn, wn[...], preferred_element_type=jnp.float32)
        + b[...]
    )
    y = jnp.maximum(y, 0.0)
    if last:
        nrm = jnp.sqrt(jnp.sum(y * y, axis=1, keepdims=True))
        y = y / jnp.maximum(nrm, 1e-12)
    out[...] = y


def _tc_layer(h, p, invb, ws, wn, b, last):
    blk = 1000
    grid = (N_NODES // blk,)
    return pl.pallas_call(
        functools.partial(_tc_layer_body, last),
        grid=grid,
        in_specs=[
            pl.BlockSpec((blk, D), lambda i: (i, 0)),
            pl.BlockSpec((NC, blk, D), lambda i: (0, i, 0)),
            pl.BlockSpec((blk, D), lambda i: (i, 0)),
            pl.BlockSpec((D, D), lambda i: (0, 0)),
            pl.BlockSpec((D, D), lambda i: (0, 0)),
            pl.BlockSpec((1, D), lambda i: (0, 0)),
        ],
        out_specs=pl.BlockSpec((blk, D), lambda i: (i, 0)),
        out_shape=jax.ShapeDtypeStruct((N_NODES, D), jnp.float32),
    )(h, p, invb, ws, wn, b)


@jax.jit
def kernel(x, edge_index, W_self1, W_neigh1, b1, W_self2, W_neigh2, b2):
    src = edge_index[0].astype(jnp.int32)
    dst = edge_index[1].astype(jnp.int32)
    pad = E_PAD - N_EDGES
    src_p = jnp.concatenate(
        [src, jnp.zeros((pad,), jnp.int32)]).reshape(E_ROWS, CHUNK)
    dst_p = jnp.concatenate(
        [dst, jnp.full((pad,), N_NODES, jnp.int32)]).reshape(E_ROWS, CHUNK)
    zcnt = jnp.zeros((R_PAD,), jnp.float32)
    zrow = jnp.zeros((ROWS_PER_TILE, D), jnp.float32)

    invb, _ = _deg_kernel(dst_p, zcnt)
    b1r = b1.reshape(1, D)
    b2r = b2.reshape(1, D)

    p1 = _agg_kernel(x, src_p, dst_p, zrow)
    h1 = _tc_layer(x, p1, invb, W_self1, W_neigh1, b1r, last=False)
    p2 = _agg_kernel(h1, src_p, dst_p, zrow)
    return _tc_layer(h1, p2, invb, W_self2, W_neigh2, b2r, last=True)


# speedup vs baseline: 1.1177x; 1.0042x over previous
import functools

import jax
import jax.numpy as jnp
from jax import lax
from jax.experimental import pallas as pl
from jax.experimental.pallas import tpu as pltpu
from jax.experimental.pallas import tpu_sc as plsc

N_NODES = 10000
D = 128
N_EDGES = 320000

NC = 2
NS = 16
NW = NC * NS

R_PAD = 10240
E_PAD = 327680
CHUNK = 128
GROUP = 8
CPW0 = 152
CPW1 = 8
NBUF = 2
E_ROWS = E_PAD // CHUNK
ROWS_PER_TILE = R_PAD // NS
INV_ROWS = R_PAD // NW
DEG_ROWS = E_ROWS // NS

_mesh = plsc.VectorSubcoreMesh(core_axis_name="c", subcore_axis_name="s")


@functools.partial(
    pl.kernel,
    out_type=[jax.ShapeDtypeStruct((R_PAD, D), jnp.float32),
              jax.ShapeDtypeStruct((NS, R_PAD), jnp.float32)],
    mesh=_mesh,
    compiler_params=pltpu.CompilerParams(needs_layout_passes=False),
    scratch_types=[
        pltpu.VMEM((R_PAD,), jnp.float32),
        pltpu.VMEM((DEG_ROWS, CHUNK), jnp.int32),
        pltpu.VMEM((NS, ROWS_PER_TILE), jnp.float32),
        pltpu.VMEM((ROWS_PER_TILE,), jnp.float32),
        pltpu.VMEM((INV_ROWS, D), jnp.float32),
    ],
)
def _deg_kernel(dst_hbm, zcnt_hbm, invb_hbm, degsh,
                cnt, dstv, deg16, totbuf, rowbuf):
    cid = lax.axis_index("c")
    sid = lax.axis_index("s")

    pltpu.sync_copy(zcnt_hbm, cnt)
    pltpu.sync_copy(dst_hbm.at[pl.ds(sid * DEG_ROWS, DEG_ROWS)], dstv)

    ones = jnp.ones((16,), jnp.float32)

    def count_body(i, carry):
        r = i // 8
        j = i % 8
        idx = dstv[r, pl.ds(j * 16, 16)]
        plsc.addupdate_scatter(cnt, [idx], ones)
        return carry

    lax.fori_loop(0, DEG_ROWS * 8, count_body, 0)

    pltpu.sync_copy(cnt, degsh.at[sid])
    plsc.subcore_barrier()

    base = sid * ROWS_PER_TILE
    for r in range(NS):
        pltpu.sync_copy(degsh.at[r, pl.ds(base, ROWS_PER_TILE)], deg16.at[r])

    def sum_body(j, carry):
        tot = deg16[0, pl.ds(j * 16, 16)]
        for r in range(1, NS):
            tot = tot + deg16[r, pl.ds(j * 16, 16)]
        totbuf[pl.ds(j * 16, 16)] = tot
        return carry

    lax.fori_loop(0, ROWS_PER_TILE // 16, sum_body, 0)

    @pl.when(cid == 0)
    def _():
        for half in range(2):
            def inv_body(g, carry):
                d16 = totbuf[pl.ds(half * INV_ROWS + g * 16, 16)]
                vinv = 1.0 / jnp.maximum(d16, 1.0)
                for l in range(16):
                    bc = jnp.full((16,), vinv[l], jnp.float32)
                    for j in range(8):
                        rowbuf[g * 16 + l, pl.ds(j * 16, 16)] = bc
                return carry

            lax.fori_loop(0, INV_ROWS // 16, inv_body, 0)
            pltpu.sync_copy(
                rowbuf,
                invb_hbm.at[pl.ds(base + half * INV_ROWS, INV_ROWS)])


@functools.partial(
    pl.kernel,
    out_type=jax.ShapeDtypeStruct((NC, R_PAD, D), jnp.float32),
    mesh=_mesh,
    scratch_types=[
        pltpu.VMEM((GROUP, CHUNK), jnp.int32),
        pltpu.VMEM((GROUP, CHUNK), jnp.int32),
        [pltpu.VMEM((CHUNK, D), jnp.float32) for _ in range(NBUF)],
        [pltpu.SemaphoreType.DMA for _ in range(NBUF)],
        pltpu.VMEM_SHARED((R_PAD, D), jnp.float32),
    ],
)
def _agg_kernel(h_hbm, src_hbm, dst_hbm, zrow_hbm, part_hbm,
                sidx, didx, bufs, gsems, acc):
    cid = lax.axis_index("c")
    sid = lax.axis_index("s")

    my_cpw = jnp.where(cid == 0, CPW0, CPW1)
    row_base = jnp.where(cid == 0, sid * CPW0, NS * CPW0 + sid * CPW1)

    pltpu.sync_copy(zrow_hbm, acc.at[pl.ds(sid * ROWS_PER_TILE, ROWS_PER_TILE)])
    plsc.subcore_barrier()

    def start_gather(c, k):
        pltpu.async_copy(h_hbm.at[sidx.at[c]], bufs[k], gsems[k])

    def wait_gather(k):
        pltpu.make_async_copy(h_hbm.at[pl.ds(0, CHUNK)], bufs[k], gsems[k]).wait()

    def group_body(g, carry):
        row0 = row_base + g * GROUP
        pltpu.sync_copy(src_hbm.at[pl.ds(row0, GROUP)], sidx)
        pltpu.sync_copy(dst_hbm.at[pl.ds(row0, GROUP)], didx)
        start_gather(0, 0)

        def body(i, carry):
            c0 = NBUF * i
            for k in range(NBUF):
                c = c0 + k

                @pl.when(c + 1 < GROUP)
                def _(c=c, k=k):
                    start_gather(c + 1, (k + 1) % NBUF)

                wait_gather(k)
                pltpu.sync_copy(bufs[k], acc.at[didx.at[c]], add=True)
            return carry

        lax.fori_loop(0, GROUP // NBUF, body, 0)
        return carry

    lax.fori_loop(0, my_cpw // GROUP, group_body, 0)
    plsc.subcore_barrier()

    pltpu.sync_copy(
        acc.at[pl.ds(sid * ROWS_PER_TILE, ROWS_PER_TILE)],
        part_hbm.at[cid, pl.ds(sid * ROWS_PER_TILE, ROWS_PER_TILE)],
    )


def _tc_layer_body(last, h, p, invb, ws, wn, b, out):
    hn = (p[0] + p[1]) * invb[...]
    y = (
        jnp.dot(h[...], ws[...], preferred_element_type=jnp.float32)
        + jnp.dot(hn, wn[...], preferred_element_type=jnp.float32)
        + b[...]
    )
    y = jnp.maximum(y, 0.0)
    if last:
        nrm = jnp.sqrt(jnp.sum(y * y, axis=1, keepdims=True))
        y = y / jnp.maximum(nrm, 1e-12)
    out[...] = y


def _tc_layer(h, p, invb, ws, wn, b, last):
    blk = 1000
    grid = (N_NODES // blk,)
    return pl.pallas_call(
        functools.partial(_tc_layer_body, last),
        grid=grid,
        in_specs=[
            pl.BlockSpec((blk, D), lambda i: (i, 0)),
            pl.BlockSpec((NC, blk, D), lambda i: (0, i, 0)),
            pl.BlockSpec((blk, D), lambda i: (i, 0)),
            pl.BlockSpec((D, D), lambda i: (0, 0)),
            pl.BlockSpec((D, D), lambda i: (0, 0)),
            pl.BlockSpec((1, D), lambda i: (0, 0)),
        ],
        out_specs=pl.BlockSpec((blk, D), lambda i: (i, 0)),
        out_shape=jax.ShapeDtypeStruct((N_NODES, D), jnp.float32),
    )(h, p, invb, ws, wn, b)


@jax.jit
def kernel(x, edge_index, W_self1, W_neigh1, b1, W_self2, W_neigh2, b2):
    src = edge_index[0].astype(jnp.int32)
    dst = edge_index[1].astype(jnp.int32)
    pad = E_PAD - N_EDGES
    src_p = jnp.concatenate(
        [src, jnp.zeros((pad,), jnp.int32)]).reshape(E_ROWS, CHUNK)
    dst_p = jnp.concatenate(
        [dst, jnp.full((pad,), N_NODES, jnp.int32)]).reshape(E_ROWS, CHUNK)
    zcnt = jnp.zeros((R_PAD,), jnp.float32)
    zrow = jnp.zeros((ROWS_PER_TILE, D), jnp.float32)

    invb, _ = _deg_kernel(dst_p, zcnt)
    b1r = b1.reshape(1, D)
    b2r = b2.reshape(1, D)

    p1 = _agg_kernel(x, src_p, dst_p, zrow)
    h1 = _tc_layer(x, p1, invb, W_self1, W_neigh1, b1r, last=False)
    p2 = _agg_kernel(h1, src_p, dst_p, zrow)
    return _tc_layer(h1, p2, invb, W_self2, W_neigh2, b2r, last=True)
